# Initial kernel scaffold; baseline (speedup 1.0000x reference)
import functools
import math

import jax
import jax.numpy as jnp
from jax import lax
from jax.experimental import pallas as pl
from jax.experimental.pallas import tpu as pltpu
from jax.experimental.pallas import tpu_sc as plsc

BATCH = 16384
HIST = 200
EMB = 32
N = BATCH * HIST
SCALE = math.sqrt(EMB)

NUM_CORES = 2
NUM_SUBCORES = 16
NW = NUM_CORES * NUM_SUBCORES
PER_W = N // NW

IDX_MINOR = 128
GATHERS_PER_CHUNK = 8
CHUNK = IDX_MINOR * GATHERS_PER_CHUNK
NCHUNKS = PER_W // CHUNK

_mesh = plsc.VectorSubcoreMesh(core_axis_name="c", subcore_axis_name="s")


@functools.partial(
    pl.kernel,
    mesh=_mesh,
    out_type=jax.ShapeDtypeStruct((N, EMB), jnp.float32),
    scratch_types=[
        pltpu.VMEM((GATHERS_PER_CHUNK, IDX_MINOR), jnp.int32),
        pltpu.VMEM((CHUNK, EMB), jnp.float32),
        pltpu.SemaphoreType.DMA,
    ],
)
def _embed_sc(tok_hbm, tab_hbm, out_hbm, idx_v, rows_v, sem):
    wid = lax.axis_index("s") * NUM_CORES + lax.axis_index("c")
    idx_row_base = wid * (PER_W // IDX_MINOR)
    out_base = wid * PER_W

    def chunk_body(ci, carry):
        pltpu.sync_copy(
            tok_hbm.at[pl.ds(idx_row_base + ci * GATHERS_PER_CHUNK,
                             GATHERS_PER_CHUNK)],
            idx_v,
        )
        copies = [
            pltpu.async_copy(
                tab_hbm.at[idx_v.at[j]],
                rows_v.at[pl.ds(j * IDX_MINOR, IDX_MINOR)],
                sem,
            )
            for j in range(GATHERS_PER_CHUNK)
        ]
        for c in copies:
            c.wait()

        def scale_body(r, c):
            for u in range(8):
                row = r * 8 + u
                for h in range(2):
                    sl = rows_v[row, pl.ds(h * 16, 16)]
                    rows_v[row, pl.ds(h * 16, 16)] = sl * SCALE
            return c

        lax.fori_loop(0, CHUNK // 8, scale_body, 0)

        pltpu.sync_copy(
            rows_v, out_hbm.at[pl.ds(out_base + ci * CHUNK, CHUNK)]
        )
        return carry

    lax.fori_loop(0, NCHUNKS, chunk_body, 0)


def kernel(tokens, table):
    tok2d = tokens.astype(jnp.int32).reshape(N // IDX_MINOR, IDX_MINOR)
    out = _embed_sc(tok2d, table)
    return out.reshape(BATCH, HIST, EMB)


# speedup vs baseline: 4.5682x; 4.5682x over previous
import functools
import math

import jax
import jax.numpy as jnp
from jax import lax
from jax.experimental import pallas as pl
from jax.experimental.pallas import tpu as pltpu
from jax.experimental.pallas import tpu_sc as plsc

BATCH = 16384
HIST = 200
EMB = 32
N = BATCH * HIST
SCALE = math.sqrt(EMB)

NUM_CORES = 2
NUM_SUBCORES = 16
NW = NUM_CORES * NUM_SUBCORES
PER_W = N // NW

IDX_MINOR = 128
GATHERS_PER_CHUNK = 8
CHUNK = IDX_MINOR * GATHERS_PER_CHUNK
NCHUNKS = PER_W // CHUNK

_mesh = plsc.VectorSubcoreMesh(core_axis_name="c", subcore_axis_name="s")


@functools.partial(
    pl.kernel,
    mesh=_mesh,
    compiler_params=pltpu.CompilerParams(use_tc_tiling_on_sc=False),
    out_type=jax.ShapeDtypeStruct((N, EMB), jnp.float32),
    scratch_types=[
        pltpu.VMEM((GATHERS_PER_CHUNK, IDX_MINOR), jnp.int32),
        pltpu.VMEM((CHUNK, EMB), jnp.float32),
        pltpu.SemaphoreType.DMA,
    ],
)
def _embed_sc(tok_hbm, tab_hbm, out_hbm, idx_v, rows_v, sem):
    wid = lax.axis_index("s") * NUM_CORES + lax.axis_index("c")
    idx_row_base = wid * (PER_W // IDX_MINOR)
    out_base = wid * PER_W

    def chunk_body(ci, carry):
        pltpu.sync_copy(
            tok_hbm.at[pl.ds(idx_row_base + ci * GATHERS_PER_CHUNK,
                             GATHERS_PER_CHUNK)],
            idx_v,
        )
        copies = [
            pltpu.async_copy(
                tab_hbm.at[idx_v.at[j]],
                rows_v.at[pl.ds(j * IDX_MINOR, IDX_MINOR)],
                sem,
            )
            for j in range(GATHERS_PER_CHUNK)
        ]
        for c in copies:
            c.wait()

        def scale_body(r, c):
            for u in range(8):
                row = r * 8 + u
                for h in range(2):
                    sl = rows_v[row, pl.ds(h * 16, 16)]
                    rows_v[row, pl.ds(h * 16, 16)] = sl * SCALE
            return c

        lax.fori_loop(0, CHUNK // 8, scale_body, 0)

        pltpu.sync_copy(
            rows_v, out_hbm.at[pl.ds(out_base + ci * CHUNK, CHUNK)]
        )
        return carry

    lax.fori_loop(0, NCHUNKS, chunk_body, 0)


def kernel(tokens, table):
    tok2d = tokens.astype(jnp.int32).reshape(N // IDX_MINOR, IDX_MINOR)
    out = _embed_sc(tok2d, table)
    return out.reshape(BATCH, HIST, EMB)


# speedup vs baseline: 5.0131x; 1.0974x over previous
import functools
import math

import jax
import jax.numpy as jnp
from jax import lax
from jax.experimental import pallas as pl
from jax.experimental.pallas import tpu as pltpu
from jax.experimental.pallas import tpu_sc as plsc

BATCH = 16384
HIST = 200
EMB = 32
N = BATCH * HIST
SCALE = math.sqrt(EMB)

NUM_CORES = 2
NUM_SUBCORES = 16
NW = NUM_CORES * NUM_SUBCORES
PER_W = N // NW

IDX_MINOR = 128
GPC = 5
CHUNK = IDX_MINOR * GPC
NCHUNKS = PER_W // CHUNK
NBUF = 4
SUPERS = NCHUNKS // NBUF

_mesh = plsc.VectorSubcoreMesh(core_axis_name="c", subcore_axis_name="s")


@functools.partial(
    pl.kernel,
    mesh=_mesh,
    compiler_params=pltpu.CompilerParams(use_tc_tiling_on_sc=False),
    out_type=jax.ShapeDtypeStruct((N, EMB), jnp.float32),
    scratch_types=[
        pltpu.VMEM((NBUF, GPC, IDX_MINOR), jnp.int32),
        pltpu.VMEM((NBUF, CHUNK, EMB), jnp.float32),
        pltpu.SemaphoreType.DMA((NBUF,)),
        pltpu.SemaphoreType.DMA((NBUF,)),
        pltpu.SemaphoreType.DMA((NBUF,)),
    ],
)
def _embed_sc(tok_hbm, tab_hbm, out_hbm, idx_v, rows_v, gsem, osem, isem):
    wid = lax.axis_index("s") * NUM_CORES + lax.axis_index("c")
    idx_row_base = wid * (PER_W // IDX_MINOR)
    out_base = wid * PER_W

    def fire_gathers(slot, chunk):
        for j in range(GPC):
            pltpu.async_copy(
                tab_hbm.at[idx_v.at[slot].at[j]],
                rows_v.at[slot].at[pl.ds(j * IDX_MINOR, IDX_MINOR)],
                gsem.at[slot],
            )

    def fire_idx(slot, chunk):
        pltpu.async_copy(
            tok_hbm.at[pl.ds(idx_row_base + chunk * GPC, GPC)],
            idx_v.at[slot],
            isem.at[slot],
        )

    def wait_gather(slot):
        pltpu.make_async_copy(
            tab_hbm.at[pl.ds(0, CHUNK)], rows_v.at[slot], gsem.at[slot]
        ).wait()

    def wait_store(slot):
        pltpu.make_async_copy(
            rows_v.at[slot], out_hbm.at[pl.ds(0, CHUNK)], osem.at[slot]
        ).wait()

    def wait_idx(slot):
        pltpu.make_async_copy(
            tok_hbm.at[pl.ds(0, GPC)], idx_v.at[slot], isem.at[slot]
        ).wait()

    def wrap(chunk):
        return lax.rem(chunk, NCHUNKS)

    for b in range(NBUF - 1):
        pltpu.sync_copy(
            tok_hbm.at[pl.ds(idx_row_base + b * GPC, GPC)], idx_v.at[b]
        )
        fire_gathers(b, b)
    fire_idx(NBUF - 1, jnp.int32(NBUF - 1))

    def super_body(k, carry):
        for b in range(NBUF):
            c = k * NBUF + b
            fb = (b + NBUF - 1) % NBUF
            wait_gather(b)

            @plsc.parallel_loop(0, CHUNK, unroll=8)
            def _scale(r):
                for h in range(EMB // 16):
                    sl = rows_v[b, r, pl.ds(h * 16, 16)]
                    rows_v[b, r, pl.ds(h * 16, 16)] = sl * SCALE

            pltpu.async_copy(
                rows_v.at[b],
                out_hbm.at[pl.ds(out_base + c * CHUNK, CHUNK)],
                osem.at[b],
            )

            wait_idx(fb)
            if b == 0:
                @pl.when(k > 0)
                def _():
                    wait_store(fb)
            else:
                wait_store(fb)
            fire_gathers(fb, wrap(c + NBUF - 1))

            fire_idx(b, wrap(c + NBUF))
        return carry

    lax.fori_loop(0, SUPERS, super_body, 0)

    for b in range(NBUF - 1):
        wait_gather(b)
    wait_idx(NBUF - 1)
    wait_store(NBUF - 1)


def kernel(tokens, table):
    tok2d = tokens.astype(jnp.int32).reshape(N // IDX_MINOR, IDX_MINOR)
    out = _embed_sc(tok2d, table)
    return out.reshape(BATCH, HIST, EMB)
